# baseline (speedup 1.0000x reference)
import functools

import jax
import jax.numpy as jnp
from jax import lax
from jax.experimental import pallas as pl
from jax.experimental.pallas import tpu as pltpu
from jax.experimental.pallas import tpu_sc as plsc

N = 10000
D_FEAT = 128
D_EDGE = 16
D_OUT = 128
CH = 128
NP = N + 240
RPT = NP // 16
NSC = 2
NSUB = 16
NW = NSC * NSUB

_MESH = plsc.VectorSubcoreMesh(core_axis_name="c", subcore_axis_name="s")
_SC_PARAMS = pltpu.CompilerParams(use_tc_tiling_on_sc=False)
_SC_PARAMS_NL = pltpu.CompilerParams(
    use_tc_tiling_on_sc=False, needs_layout_passes=False)


def _drain_shared_slice(shared, stage, out_ref, cid, row0):
    nfull, rem = RPT // CH, RPT % CH
    for j in range(nfull):
        pltpu.sync_copy(shared.at[pl.ds(row0 + j * CH, CH), :], stage)
        pltpu.sync_copy(stage, out_ref.at[cid, pl.ds(row0 + j * CH, CH), :])
    if rem:
        pltpu.sync_copy(
            shared.at[pl.ds(row0 + nfull * CH, rem), :],
            stage.at[pl.ds(0, rem), :],
        )
        pltpu.sync_copy(
            stage.at[pl.ds(0, rem), :],
            out_ref.at[cid, pl.ds(row0 + nfull * CH, rem), :],
        )


IDXR = 12
RPT_E = 156


def _deg_body(edge_ref, out_ref, stage_sh, cnt_v, ibuf_v, red_v, st2_v,
              seml0, seml1):
    cid = lax.axis_index("c")
    sid = lax.axis_index("s")
    nrow = edge_ref.shape[0]
    base = sid * RPT_E
    seml = (seml0, seml1)

    def zero(i, carry):
        cnt_v[pl.ds(i * 16, 16)] = jnp.zeros((16,), jnp.float32)
        return carry

    lax.fori_loop(0, NP // 16, zero, 0)

    nchunk = RPT_E // IDXR

    def issue(k, s):
        pltpu.async_copy(edge_ref.at[pl.ds(base + k * IDXR, IDXR), :, :],
                         ibuf_v.at[s], seml[s])

    def wait(s):
        pltpu.make_async_copy(edge_ref.at[pl.ds(0, IDXR), :, :],
                              ibuf_v.at[s], seml[s]).wait()

    def hist16(idx):
        cnts, last = plsc.scan_count(idx)
        plsc.addupdate_scatter(
            cnt_v, [idx], cnts.astype(jnp.float32), mask=last)

    issue(0, 0)
    for k in range(nchunk):
        s = k % 2
        wait(s)
        if k + 1 < nchunk:
            issue(k + 1, 1 - s)

        def group(g, carry):
            for u in range(4):
                gg = g * 4 + u
                row = gg // 8
                grp = gg % 8
                hist16(ibuf_v[s, row, cid, pl.ds(grp * 16, 16)])
            return carry

        lax.fori_loop(0, IDXR * 8 // 4, group, 0)

    @pl.when(sid < nrow - NSUB * RPT_E)
    def _():
        pltpu.sync_copy(edge_ref.at[pl.ds(NSUB * RPT_E + sid, 1), :, :],
                        ibuf_v.at[0, pl.ds(0, 1), :, :])
        for grp in range(8):
            hist16(ibuf_v[0, 0, cid, pl.ds(grp * 16, 16)])

    pltpu.sync_copy(cnt_v, stage_sh.at[sid])
    plsc.subcore_barrier()

    for t in range(NSUB):
        pltpu.sync_copy(stage_sh.at[t, pl.ds(sid * RPT, RPT)], red_v.at[t])

    lanes = lax.iota(jnp.int32, 16)
    zeros16 = jnp.zeros((16,), jnp.int32)

    def red(g, carry):
        acc = red_v[0, pl.ds(g * 16, 16)]
        for t in range(1, NSUB):
            acc = acc + red_v[t, pl.ds(g * 16, 16)]
        plsc.store_scatter(st2_v, [g * 16 + lanes, zeros16], acc)
        return carry

    lax.fori_loop(0, RPT // 16, red, 0)
    pltpu.sync_copy(st2_v, out_ref.at[cid, pl.ds(sid * RPT, RPT), :])


@jax.jit
def _deg_sc(edge3):
    return pl.kernel(
        _deg_body,
        out_type=jax.ShapeDtypeStruct((NSC, NP, 16), jnp.float32),
        mesh=_MESH,
        scratch_types=[
            pltpu.VMEM_SHARED((NSUB, NP), jnp.float32),
            pltpu.VMEM((NP,), jnp.float32),
            pltpu.VMEM((2, IDXR, 2, CH), jnp.int32),
            pltpu.VMEM((NSUB, RPT), jnp.float32),
            pltpu.VMEM((RPT, 16), jnp.float32),
            pltpu.SemaphoreType.DMA,
            pltpu.SemaphoreType.DMA,
        ],
        compiler_params=_SC_PARAMS_NL,
    )(edge3)


def _h_body(deg_ref, feat_ref, wf_ref, h_ref):
    d = deg_ref[0, :, 0:1]
    nd = lax.rsqrt(jnp.maximum(d, 1.0))
    h_ref[...] = jnp.dot(
        feat_ref[...] * nd, wf_ref[...], preferred_element_type=jnp.float32
    )


@jax.jit
def _h_tc(deg, feat, wf):
    blk = 1000
    return pl.pallas_call(
        _h_body,
        grid=(N // blk,),
        in_specs=[
            pl.BlockSpec((1, blk, 16), lambda i: (0, i, 0)),
            pl.BlockSpec((blk, D_FEAT), lambda i: (i, 0)),
            pl.BlockSpec((D_FEAT, D_OUT), lambda i: (0, 0)),
        ],
        out_specs=pl.BlockSpec((blk, D_OUT), lambda i: (i, 0)),
        out_shape=jax.ShapeDtypeStruct((N, D_OUT), jnp.float32),
    )(deg, feat, wf)


SLOT_E = CH
SLOTS_PW = 78
LEFT_ROWS = 4


def _agg_body(h_ref, edge_ref, ef_ref, zh_ref, ze_ref, aggh_out, agge_out,
              aggh_sh, agge_sh, idxs_v, idxd_v, rows_v, erows_v,
              seml0, seml1, semg0, semg1, sems0, sems1):
    cid = lax.axis_index("c")
    sid = lax.axis_index("s")
    wid = cid * NSUB + sid
    base = wid * SLOTS_PW * SLOT_E

    seml = (seml0, seml1)
    semg = (semg0, semg1)
    sems = (sems0, sems1)

    row0 = sid * RPT
    pltpu.sync_copy(zh_ref, aggh_sh.at[pl.ds(row0, RPT), :])
    pltpu.sync_copy(ze_ref, agge_sh.at[pl.ds(row0, RPT), :])
    plsc.subcore_barrier()

    def issue_load(j, s):
        row = wid * SLOTS_PW + j
        eoff = base + j * SLOT_E
        pltpu.async_copy(edge_ref.at[row, 0, :], idxs_v.at[s], seml[s])
        pltpu.async_copy(edge_ref.at[row, 1, :], idxd_v.at[s, 0], seml[s])
        pltpu.async_copy(ef_ref.at[pl.ds(eoff, SLOT_E), :], erows_v.at[s], seml[s])

    def wait_load(s):
        pltpu.make_async_copy(edge_ref.at[0, 0, :], idxs_v.at[s], seml[s]).wait()
        pltpu.make_async_copy(edge_ref.at[0, 0, :], idxd_v.at[s, 0], seml[s]).wait()
        pltpu.make_async_copy(ef_ref.at[pl.ds(0, SLOT_E), :], erows_v.at[s], seml[s]).wait()

    def issue_gather(s):
        pltpu.async_copy(h_ref.at[idxs_v.at[s]], rows_v.at[s], semg[s])

    def wait_gather(s):
        pltpu.make_async_copy(h_ref.at[idxs_v.at[s]], rows_v.at[s], semg[s]).wait()

    def issue_scatter(s):
        pltpu.async_copy(rows_v.at[s], aggh_sh.at[idxd_v.at[s, 0]], sems[s], add=True)
        pltpu.async_copy(erows_v.at[s], agge_sh.at[idxd_v.at[s, 0]], sems[s], add=True)

    def wait_scatter(s):
        pltpu.make_async_copy(rows_v.at[s], aggh_sh.at[idxd_v.at[s, 0]], sems[s]).wait()
        pltpu.make_async_copy(erows_v.at[s], agge_sh.at[idxd_v.at[s, 0]], sems[s]).wait()

    issue_load(0, 0)
    issue_load(1, 1)
    wait_load(0)
    issue_gather(0)
    last_it = SLOTS_PW // 2 - 1

    def body(it, carry):
        wait_gather(0)
        issue_scatter(0)
        wait_load(1)
        issue_gather(1)
        wait_scatter(0)

        @pl.when(it < last_it)
        def _():
            issue_load(2 * it + 2, 0)

        wait_gather(1)
        issue_scatter(1)

        @pl.when(it < last_it)
        def _():
            wait_load(0)
            issue_gather(0)

        wait_scatter(1)

        @pl.when(it < last_it)
        def _():
            issue_load(2 * it + 3, 1)

        return carry

    lax.fori_loop(0, SLOTS_PW // 2, body, 0)

    @pl.when(wid < LEFT_ROWS)
    def _():
        row = SLOTS_PW * NW + wid
        eoff = SLOTS_PW * SLOT_E * NW + wid * CH
        pltpu.sync_copy(edge_ref.at[row, 0, :], idxs_v.at[0])
        pltpu.sync_copy(edge_ref.at[row, 1, :], idxd_v.at[0, 0])
        pltpu.sync_copy(ef_ref.at[pl.ds(eoff, CH), :], erows_v.at[0])
        pltpu.async_copy(h_ref.at[idxs_v.at[0]], rows_v.at[0], semg0).wait()
        pltpu.sync_copy(rows_v.at[0], aggh_sh.at[idxd_v.at[0, 0]], add=True)
        pltpu.sync_copy(erows_v.at[0], agge_sh.at[idxd_v.at[0, 0]], add=True)

    plsc.subcore_barrier()

    _drain_shared_slice(aggh_sh, rows_v.at[0], aggh_out, cid, row0)
    _drain_shared_slice(agge_sh, erows_v.at[0], agge_out, cid, row0)


@jax.jit
def _agg_sc(h, edge3, edge_feat, zh, ze):
    return pl.kernel(
        _agg_body,
        out_type=[
            jax.ShapeDtypeStruct((NSC, NP, D_FEAT), jnp.float32),
            jax.ShapeDtypeStruct((NSC, NP, D_EDGE), jnp.float32),
        ],
        mesh=_MESH,
        scratch_types=[
            pltpu.VMEM_SHARED((NP, D_FEAT), jnp.float32),
            pltpu.VMEM_SHARED((NP, D_EDGE), jnp.float32),
            pltpu.VMEM((2, SLOT_E), jnp.int32),
            pltpu.VMEM((2, 1, CH), jnp.int32),
            pltpu.VMEM((2, SLOT_E, D_FEAT), jnp.float32),
            pltpu.VMEM((2, SLOT_E, D_EDGE), jnp.float32),
            pltpu.SemaphoreType.DMA,
            pltpu.SemaphoreType.DMA,
            pltpu.SemaphoreType.DMA,
            pltpu.SemaphoreType.DMA,
            pltpu.SemaphoreType.DMA,
            pltpu.SemaphoreType.DMA,
        ],
        compiler_params=_SC_PARAMS,
    )(h, edge3, edge_feat, zh, ze)


def _final_body(aggh_ref, agge_ref, deg_ref, we_ref, bias_ref, out_ref):
    s = aggh_ref[0] + aggh_ref[1]
    e = agge_ref[0] + agge_ref[1]
    nd = lax.rsqrt(jnp.maximum(deg_ref[0, :, 0:1], 1.0))
    r = s + jnp.dot(e, we_ref[...], preferred_element_type=jnp.float32)
    out_ref[...] = r * nd + bias_ref[...]


@jax.jit
def _final_tc(aggh, agge, deg, we, bias):
    blk = 1000
    return pl.pallas_call(
        _final_body,
        grid=(N // blk,),
        in_specs=[
            pl.BlockSpec((NSC, blk, D_FEAT), lambda i: (0, i, 0)),
            pl.BlockSpec((NSC, blk, D_EDGE), lambda i: (0, i, 0)),
            pl.BlockSpec((1, blk, 16), lambda i: (1, i, 0)),
            pl.BlockSpec((D_EDGE, D_OUT), lambda i: (0, 0)),
            pl.BlockSpec((1, D_OUT), lambda i: (0, 0)),
        ],
        out_specs=pl.BlockSpec((blk, D_OUT), lambda i: (i, 0)),
        out_shape=jax.ShapeDtypeStruct((N, D_OUT), jnp.float32),
    )(aggh, agge, deg, we, bias[None, :])


def kernel(feat, edge_index, edge_feat, weight, bias):
    e = edge_index.shape[1]
    edge3 = edge_index.reshape(2, e // CH, CH).transpose(1, 0, 2)
    wf = weight[:D_FEAT]
    we = weight[D_FEAT:]
    zh = jnp.zeros((RPT, D_FEAT), jnp.float32)
    ze = jnp.zeros((RPT, D_EDGE), jnp.float32)
    deg = _deg_sc(edge3)
    h = _h_tc(deg, feat, wf)
    aggh, agge = _agg_sc(h, edge3, edge_feat, zh, ze)
    return _final_tc(aggh, agge, deg, we, bias)


# speedup vs baseline: 1.1046x; 1.1046x over previous
import functools

import jax
import jax.numpy as jnp
from jax import lax
from jax.experimental import pallas as pl
from jax.experimental.pallas import tpu as pltpu
from jax.experimental.pallas import tpu_sc as plsc

N = 10000
D_FEAT = 128
D_EDGE = 16
D_OUT = 128
CH = 128
NP = N + 240
RPT = NP // 16
NSC = 2
NSUB = 16
NW = NSC * NSUB

_MESH = plsc.VectorSubcoreMesh(core_axis_name="c", subcore_axis_name="s")
_SC_PARAMS = pltpu.CompilerParams(use_tc_tiling_on_sc=False)
_SC_PARAMS_NL = pltpu.CompilerParams(
    use_tc_tiling_on_sc=False, needs_layout_passes=False)


def _drain_shared_slice(shared, stage, out_ref, cid, row0, rpt):
    nfull, rem = rpt // CH, rpt % CH
    for j in range(nfull):
        pltpu.sync_copy(shared.at[pl.ds(row0 + j * CH, CH), :], stage)
        pltpu.sync_copy(stage, out_ref.at[cid, pl.ds(row0 + j * CH, CH), :])
    if rem:
        pltpu.sync_copy(
            shared.at[pl.ds(row0 + nfull * CH, rem), :],
            stage.at[pl.ds(0, rem), :],
        )
        pltpu.sync_copy(
            stage.at[pl.ds(0, rem), :],
            out_ref.at[cid, pl.ds(row0 + nfull * CH, rem), :],
        )


IDXR = 12
RPT_E = 156


def _deg_body(edge_ref, out_ref, stage_sh, cnt_v, ibuf_v, red_v, st2_v,
              seml0, seml1):
    cid = lax.axis_index("c")
    sid = lax.axis_index("s")
    nrow = edge_ref.shape[0]
    base = sid * RPT_E
    seml = (seml0, seml1)

    def zero(i, carry):
        cnt_v[pl.ds(i * 16, 16)] = jnp.zeros((16,), jnp.float32)
        return carry

    lax.fori_loop(0, NP // 16, zero, 0)

    nchunk = RPT_E // IDXR

    def issue(k, s):
        pltpu.async_copy(edge_ref.at[pl.ds(base + k * IDXR, IDXR), :, :],
                         ibuf_v.at[s], seml[s])

    def wait(s):
        pltpu.make_async_copy(edge_ref.at[pl.ds(0, IDXR), :, :],
                              ibuf_v.at[s], seml[s]).wait()

    def hist16(idx):
        cnts, last = plsc.scan_count(idx)
        plsc.addupdate_scatter(
            cnt_v, [idx], cnts.astype(jnp.float32), mask=last)

    issue(0, 0)
    for k in range(nchunk):
        s = k % 2
        wait(s)
        if k + 1 < nchunk:
            issue(k + 1, 1 - s)

        def group(g, carry):
            for u in range(4):
                gg = g * 4 + u
                row = gg // 8
                grp = gg % 8
                hist16(ibuf_v[s, row, cid, pl.ds(grp * 16, 16)])
            return carry

        lax.fori_loop(0, IDXR * 8 // 4, group, 0)

    @pl.when(sid < nrow - NSUB * RPT_E)
    def _():
        pltpu.sync_copy(edge_ref.at[pl.ds(NSUB * RPT_E + sid, 1), :, :],
                        ibuf_v.at[0, pl.ds(0, 1), :, :])
        for grp in range(8):
            hist16(ibuf_v[0, 0, cid, pl.ds(grp * 16, 16)])

    pltpu.sync_copy(cnt_v, stage_sh.at[sid])
    plsc.subcore_barrier()

    for t in range(NSUB):
        pltpu.sync_copy(stage_sh.at[t, pl.ds(sid * RPT, RPT)], red_v.at[t])

    lanes = lax.iota(jnp.int32, 16)
    zeros16 = jnp.zeros((16,), jnp.int32)

    def red(g, carry):
        acc = red_v[0, pl.ds(g * 16, 16)]
        for t in range(1, NSUB):
            acc = acc + red_v[t, pl.ds(g * 16, 16)]
        plsc.store_scatter(st2_v, [g * 16 + lanes, zeros16], acc)
        return carry

    lax.fori_loop(0, RPT // 16, red, 0)
    pltpu.sync_copy(st2_v, out_ref.at[cid, pl.ds(sid * RPT, RPT), :])


@jax.jit
def _deg_sc(edge3):
    return pl.kernel(
        _deg_body,
        out_type=jax.ShapeDtypeStruct((NSC, NP, 16), jnp.float32),
        mesh=_MESH,
        scratch_types=[
            pltpu.VMEM_SHARED((NSUB, NP), jnp.float32),
            pltpu.VMEM((NP,), jnp.float32),
            pltpu.VMEM((2, IDXR, 2, CH), jnp.int32),
            pltpu.VMEM((NSUB, RPT), jnp.float32),
            pltpu.VMEM((RPT, 16), jnp.float32),
            pltpu.SemaphoreType.DMA,
            pltpu.SemaphoreType.DMA,
        ],
        compiler_params=_SC_PARAMS_NL,
    )(edge3)


def _h_body(deg_ref, feat_ref, wf_ref, h_ref):
    d = deg_ref[0, :, 0:1]
    nd = lax.rsqrt(jnp.maximum(d, 1.0))
    h_ref[...] = jnp.dot(
        feat_ref[...] * nd, wf_ref[...], preferred_element_type=jnp.float32
    )


@jax.jit
def _h_tc(deg, feat, wf):
    blk = 1000
    return pl.pallas_call(
        _h_body,
        grid=(N // blk,),
        in_specs=[
            pl.BlockSpec((1, blk, 16), lambda i: (0, i, 0)),
            pl.BlockSpec((blk, D_FEAT), lambda i: (i, 0)),
            pl.BlockSpec((D_FEAT, D_OUT), lambda i: (0, 0)),
        ],
        out_specs=pl.BlockSpec((blk, D_OUT), lambda i: (i, 0)),
        out_shape=jax.ShapeDtypeStruct((N, D_OUT), jnp.float32),
    )(deg, feat, wf)


SLOT_E = CH
SLOTS_PW = 78
LEFT_ROWS = 4


RPTS = N // NSUB


def _agg_body(h_ref, edge_ref, ef_ref, zh_ref, ze_ref, aggh_out, agge_out,
              aggh_sh, agge_sh, idxs_v, idxd_v, rows_v, erows_v, etmp_v,
              seml0, seml1, semg0, semg1, sems0, sems1):
    cid = lax.axis_index("c")
    sid = lax.axis_index("s")
    wid = cid * NSUB + sid

    seml = (seml0, seml1)
    semg = (semg0, semg1)
    sems = (sems0, sems1)

    row0 = sid * RPTS
    pltpu.sync_copy(zh_ref, aggh_sh.at[pl.ds(row0, RPTS), :])
    pltpu.sync_copy(ze_ref, agge_sh.at[pl.ds(row0, RPTS), :])
    plsc.subcore_barrier()

    lanes = lax.iota(jnp.int32, 16)
    c8v = lanes // 8
    clv = lanes % 8

    def issue_load(j, s):
        row = wid * SLOTS_PW + j
        pltpu.async_copy(edge_ref.at[row, 0, :], idxs_v.at[s], seml[s])
        pltpu.async_copy(edge_ref.at[row, 1, :], idxd_v.at[s, 0], seml[s])
        pltpu.async_copy(ef_ref.at[0, row, :, :], etmp_v.at[s, 0], seml[s])
        pltpu.async_copy(ef_ref.at[1, row, :, :], etmp_v.at[s, 1], seml[s])

    def wait_load(s):
        pltpu.make_async_copy(edge_ref.at[0, 0, :], idxs_v.at[s], seml[s]).wait()
        pltpu.make_async_copy(edge_ref.at[0, 0, :], idxd_v.at[s, 0], seml[s]).wait()
        pltpu.make_async_copy(ef_ref.at[0, 0, :, :], etmp_v.at[s, 0], seml[s]).wait()
        pltpu.make_async_copy(ef_ref.at[0, 0, :, :], etmp_v.at[s, 1], seml[s]).wait()

    def issue_gather(s):
        pltpu.async_copy(h_ref.at[idxs_v.at[s]], rows_v.at[s], semg[s])

    def wait_gather(s):
        pltpu.make_async_copy(h_ref.at[idxs_v.at[s]], rows_v.at[s], semg[s]).wait()

    def transpose_ef(s):
        def body(e, carry):
            for u in range(4):
                ev = jnp.zeros((16,), jnp.int32) + (e * 4 + u)
                vals = plsc.load_gather(etmp_v.at[s], [c8v, clv, ev])
                erows_v[e * 4 + u, :] = vals
            return carry

        lax.fori_loop(0, CH // 4, body, 0)

    def issue_scatter(s):
        pltpu.async_copy(rows_v.at[s], aggh_sh.at[idxd_v.at[s, 0]], sems[s], add=True)
        pltpu.async_copy(erows_v, agge_sh.at[idxd_v.at[s, 0]], sems[s], add=True)

    def wait_scatter(s):
        pltpu.make_async_copy(rows_v.at[s], aggh_sh.at[idxd_v.at[s, 0]], sems[s]).wait()
        pltpu.make_async_copy(erows_v, agge_sh.at[idxd_v.at[s, 0]], sems[s]).wait()

    issue_load(0, 0)
    issue_load(1, 1)
    wait_load(0)
    issue_gather(0)
    last_it = SLOTS_PW // 2 - 1

    def half(s, it, j):
        transpose_ef(s)
        wait_gather(s)
        issue_scatter(s)
        if s == 0:
            wait_load(1)
            issue_gather(1)
        else:
            @pl.when(it < last_it)
            def _():
                wait_load(0)
                issue_gather(0)
        wait_scatter(s)

        @pl.when(it < last_it)
        def _():
            issue_load(j + 2, s)

    def body(it, carry):
        half(0, it, 2 * it)
        half(1, it, 2 * it + 1)
        return carry

    lax.fori_loop(0, SLOTS_PW // 2, body, 0)

    @pl.when(wid < LEFT_ROWS)
    def _():
        row = SLOTS_PW * NW + wid
        pltpu.sync_copy(edge_ref.at[row, 0, :], idxs_v.at[0])
        pltpu.sync_copy(edge_ref.at[row, 1, :], idxd_v.at[0, 0])
        pltpu.sync_copy(ef_ref.at[0, row, :, :], etmp_v.at[0, 0])
        pltpu.sync_copy(ef_ref.at[1, row, :, :], etmp_v.at[0, 1])
        transpose_ef(0)
        pltpu.async_copy(h_ref.at[idxs_v.at[0]], rows_v.at[0], semg0).wait()
        pltpu.sync_copy(rows_v.at[0], aggh_sh.at[idxd_v.at[0, 0]], add=True)
        pltpu.sync_copy(erows_v, agge_sh.at[idxd_v.at[0, 0]], add=True)

    plsc.subcore_barrier()

    _drain_shared_slice(aggh_sh, rows_v.at[0], aggh_out, cid, row0, RPTS)
    _drain_shared_slice(agge_sh, erows_v, agge_out, cid, row0, RPTS)


@jax.jit
def _agg_sc(h, edge3, ef4, zh, ze):
    return pl.kernel(
        _agg_body,
        out_type=[
            jax.ShapeDtypeStruct((NSC, N, D_FEAT), jnp.float32),
            jax.ShapeDtypeStruct((NSC, N, D_EDGE), jnp.float32),
        ],
        mesh=_MESH,
        scratch_types=[
            pltpu.VMEM_SHARED((N, D_FEAT), jnp.float32),
            pltpu.VMEM_SHARED((N, D_EDGE), jnp.float32),
            pltpu.VMEM((2, SLOT_E), jnp.int32),
            pltpu.VMEM((2, 1, CH), jnp.int32),
            pltpu.VMEM((2, SLOT_E, D_FEAT), jnp.float32),
            pltpu.VMEM((SLOT_E, D_EDGE), jnp.float32),
            pltpu.VMEM((2, 2, 8, CH), jnp.float32),
            pltpu.SemaphoreType.DMA,
            pltpu.SemaphoreType.DMA,
            pltpu.SemaphoreType.DMA,
            pltpu.SemaphoreType.DMA,
            pltpu.SemaphoreType.DMA,
            pltpu.SemaphoreType.DMA,
        ],
        compiler_params=_SC_PARAMS_NL,
    )(h, edge3, ef4, zh, ze)


def _final_body(aggh_ref, agge_ref, deg_ref, we_ref, bias_ref, out_ref):
    s = aggh_ref[0] + aggh_ref[1]
    e = agge_ref[0] + agge_ref[1]
    nd = lax.rsqrt(jnp.maximum(deg_ref[0, :, 0:1], 1.0))
    r = s + jnp.dot(e, we_ref[...], preferred_element_type=jnp.float32)
    out_ref[...] = r * nd + bias_ref[...]


@jax.jit
def _final_tc(aggh, agge, deg, we, bias):
    blk = 1000
    return pl.pallas_call(
        _final_body,
        grid=(N // blk,),
        in_specs=[
            pl.BlockSpec((NSC, blk, D_FEAT), lambda i: (0, i, 0)),
            pl.BlockSpec((NSC, blk, D_EDGE), lambda i: (0, i, 0)),
            pl.BlockSpec((1, blk, 16), lambda i: (1, i, 0)),
            pl.BlockSpec((D_EDGE, D_OUT), lambda i: (0, 0)),
            pl.BlockSpec((1, D_OUT), lambda i: (0, 0)),
        ],
        out_specs=pl.BlockSpec((blk, D_OUT), lambda i: (i, 0)),
        out_shape=jax.ShapeDtypeStruct((N, D_OUT), jnp.float32),
    )(aggh, agge, deg, we, bias[None, :])


def kernel(feat, edge_index, edge_feat, weight, bias):
    e = edge_index.shape[1]
    edge3 = edge_index.reshape(2, e // CH, CH).transpose(1, 0, 2)
    ef4 = edge_feat.T.reshape(2, 8, e // CH, CH).transpose(0, 2, 1, 3)
    wf = weight[:D_FEAT]
    we = weight[D_FEAT:]
    zh = jnp.zeros((N // NSUB, D_FEAT), jnp.float32)
    ze = jnp.zeros((N // NSUB, D_EDGE), jnp.float32)
    deg = _deg_sc(edge3)
    h = _h_tc(deg, feat, wf)
    aggh, agge = _agg_sc(h, edge3, ef4, zh, ze)
    return _final_tc(aggh, agge, deg, we, bias)
